# Initial kernel scaffold; baseline (speedup 1.0000x reference)
import jax
import jax.numpy as jnp
from jax.experimental import pallas as pl


def kernel(x, W_in, W1q, W1n, b1, W2, b2, W_out, b_out):
    raise NotImplementedError("write your pallas kernel here")


# speedup vs baseline: 10.3695x; 10.3695x over previous
import functools

import numpy as np
import jax
import jax.numpy as jnp
from jax.experimental import pallas as pl
from jax.experimental.pallas import tpu as pltpu

B, S, D, H, K = 1, 2048, 1024, 16, 32
DH = D // H
LEVELS = 12
T = 128
NT = S // T
W = 34
NO = 2 * W + 1
S_PAD = 2176


def _routes_np():
    n, k = S, K
    t = ((np.arange(n, dtype=np.float32) + np.float32(0.5)) / np.float32(n)).astype(np.float32)
    c = np.zeros(n, dtype=np.float32)
    frac = t
    stopped = np.zeros(n, dtype=bool)
    for l in range(LEVELS):
        d = np.clip(np.floor(frac * np.float32(3.0)).astype(np.int32), 0, 2)
        frac = (frac * np.float32(3.0) - d.astype(np.float32)).astype(np.float32)
        scale = np.float32(0.5 ** (l + 1))
        add = np.where(d == 1, np.float32(1.0), d.astype(np.float32) * np.float32(0.5)) * scale
        c = (c + np.where(stopped, np.float32(0.0), add).astype(np.float32)).astype(np.float32)
        stopped = stopped | (d == 1)
    pos = np.arange(n, dtype=np.float32)
    dist = (np.abs(c[:, None] - c[None, :]).astype(np.float32)
            + (np.abs(pos[:, None] - pos[None, :]) / np.float32(n * 1e6)).astype(np.float32))
    return np.argsort(dist.astype(np.float32), axis=-1, kind="stable")[:, :k].astype(np.int32)


@functools.lru_cache(maxsize=1)
def _static_tables():
    routes = _routes_np()
    off = routes - np.arange(S, dtype=np.int32)[:, None]
    assert np.abs(off).max() <= W
    inv = np.ones((S, NO), dtype=np.float32)
    np.put_along_axis(inv, off + W, 0.0, axis=1)
    inv = inv.reshape(NT, T, NO)
    rep = np.kron(np.eye(H, dtype=np.float32), np.ones((1, DH), np.float32))
    return inv, rep


def _proj_kernel(x_ref, win_ref, bdq_ref, bdn_ref, b1_ref,
                 h_ref, q_ref, n_ref):
    xv = x_ref[...]
    h = jnp.dot(xv, win_ref[...], preferred_element_type=jnp.float32)
    h_ref[...] = h
    q_ref[...] = jnp.dot(h, bdq_ref[...], preferred_element_type=jnp.float32) + b1_ref[...]
    n_ref[...] = jnp.dot(h, bdn_ref[...], preferred_element_type=jnp.float32)


def _fuse_kernel(q_ref, x_ref, nb0_ref, nb1_ref, hb0_ref, hb1_ref, m2_ref,
                 wo_ref, w2_ref, r_ref, bo_ref, out_ref, z_scr):
    qv = q_ref[...]
    m2 = m2_ref[0]
    nwin = jnp.concatenate([nb0_ref[...], nb1_ref[...]], axis=0)

    m = jnp.full((T, H), -1e30, jnp.float32)
    for o in range(NO):
        t = jnp.maximum(qv + nwin[o:o + T], 0.0)
        z = jnp.dot(t, w2_ref[...], preferred_element_type=jnp.float32)
        z = z + jnp.float32(-1e30) * m2[:, o:o + 1]
        z_scr[o] = z
        m = jnp.maximum(m, z)

    hwin = jnp.concatenate([hb0_ref[...], hb1_ref[...]], axis=0)
    acc = jnp.zeros((T, D), jnp.float32)
    l = jnp.zeros((T, H), jnp.float32)
    for o in range(NO):
        e = jnp.exp(z_scr[o] - m)
        pr = jnp.dot(e, r_ref[...], preferred_element_type=jnp.float32)
        acc = acc + pr * hwin[o:o + T]
        l = l + e

    lrep = jnp.dot(l, r_ref[...], preferred_element_type=jnp.float32)
    fused = acc / lrep
    out = jnp.dot(fused, wo_ref[...], preferred_element_type=jnp.float32)
    out_ref[...] = out + bo_ref[...] + x_ref[...]


def kernel(x, W_in, W1q, W1n, b1, W2, b2, W_out, b_out):
    inv_np, rep_np = _static_tables()
    x2 = x.reshape(S, D)
    eye = jnp.eye(H, dtype=jnp.float32)
    bdq = jnp.kron(eye, W1q)
    bdn = jnp.kron(eye, W1n)
    w2bd = jnp.kron(eye, W2)
    b1t = jnp.tile(b1, H).reshape(1, D)
    bo = (b_out + b2[0] * 0.0).reshape(1, D)
    m2 = jnp.asarray(inv_np)
    rep = jnp.asarray(rep_np)

    h, q, n = pl.pallas_call(
        _proj_kernel,
        grid=(NT,),
        in_specs=[
            pl.BlockSpec((T, D), lambda i: (i, 0)),
            pl.BlockSpec((D, D), lambda i: (0, 0)),
            pl.BlockSpec((D, D), lambda i: (0, 0)),
            pl.BlockSpec((D, D), lambda i: (0, 0)),
            pl.BlockSpec((1, D), lambda i: (0, 0)),
        ],
        out_specs=[
            pl.BlockSpec((T, D), lambda i: (i, 0)),
            pl.BlockSpec((T, D), lambda i: (i, 0)),
            pl.BlockSpec((T, D), lambda i: (i, 0)),
        ],
        out_shape=[jax.ShapeDtypeStruct((S, D), jnp.float32)] * 3,
    )(x2, W_in, bdq, bdn, b1t)

    h_pad = jnp.pad(h, ((W, S_PAD - S - W), (0, 0)))
    n_pad = jnp.pad(n, ((W, S_PAD - S - W), (0, 0)))

    out2 = pl.pallas_call(
        _fuse_kernel,
        grid=(NT,),
        in_specs=[
            pl.BlockSpec((T, D), lambda i: (i, 0)),
            pl.BlockSpec((T, D), lambda i: (i, 0)),
            pl.BlockSpec((T, D), lambda i: (i, 0)),
            pl.BlockSpec((T, D), lambda i: (i + 1, 0)),
            pl.BlockSpec((T, D), lambda i: (i, 0)),
            pl.BlockSpec((T, D), lambda i: (i + 1, 0)),
            pl.BlockSpec((1, T, NO), lambda i: (i, 0, 0)),
            pl.BlockSpec((D, D), lambda i: (0, 0)),
            pl.BlockSpec((D, H), lambda i: (0, 0)),
            pl.BlockSpec((H, D), lambda i: (0, 0)),
            pl.BlockSpec((1, D), lambda i: (0, 0)),
        ],
        out_specs=pl.BlockSpec((T, D), lambda i: (i, 0)),
        out_shape=jax.ShapeDtypeStruct((S, D), jnp.float32),
        scratch_shapes=[
            pltpu.VMEM((NO, T, H), jnp.float32),
        ],
    )(q, x2, n_pad, n_pad, h_pad, h_pad, m2, W_out, w2bd, rep, bo)

    return out2.reshape(B, S, D)


# speedup vs baseline: 10.7489x; 1.0366x over previous
import functools

import numpy as np
import jax
import jax.numpy as jnp
from jax.experimental import pallas as pl
from jax.experimental.pallas import tpu as pltpu

B, S, D, H, K = 1, 2048, 1024, 16, 32
DH = D // H
LEVELS = 12
T = 128
NT = S // T
W = 34
NO = 2 * W + 1
S_PAD = 2176


def _routes_np():
    n, k = S, K
    t = ((np.arange(n, dtype=np.float32) + np.float32(0.5)) / np.float32(n)).astype(np.float32)
    c = np.zeros(n, dtype=np.float32)
    frac = t
    stopped = np.zeros(n, dtype=bool)
    for l in range(LEVELS):
        d = np.clip(np.floor(frac * np.float32(3.0)).astype(np.int32), 0, 2)
        frac = (frac * np.float32(3.0) - d.astype(np.float32)).astype(np.float32)
        scale = np.float32(0.5 ** (l + 1))
        add = np.where(d == 1, np.float32(1.0), d.astype(np.float32) * np.float32(0.5)) * scale
        c = (c + np.where(stopped, np.float32(0.0), add).astype(np.float32)).astype(np.float32)
        stopped = stopped | (d == 1)
    pos = np.arange(n, dtype=np.float32)
    dist = (np.abs(c[:, None] - c[None, :]).astype(np.float32)
            + (np.abs(pos[:, None] - pos[None, :]) / np.float32(n * 1e6)).astype(np.float32))
    return np.argsort(dist.astype(np.float32), axis=-1, kind="stable")[:, :k].astype(np.int32)


@functools.lru_cache(maxsize=1)
def _static_tables():
    routes = _routes_np()
    off = routes - np.arange(S, dtype=np.int32)[:, None]
    assert np.abs(off).max() <= W
    inv = np.ones((S, NO), dtype=np.float32)
    np.put_along_axis(inv, off + W, 0.0, axis=1)
    inv = inv.reshape(NT, T, NO)
    rep = np.kron(np.eye(H, dtype=np.float32), np.ones((1, DH), np.float32))
    return inv, rep


def _proj_kernel(x_ref, win_ref, bdq_ref, bdn_ref, b1_ref,
                 h_ref, q_ref, n_ref):
    xv = x_ref[...]
    h = jnp.dot(xv, win_ref[...], preferred_element_type=jnp.float32)
    h_ref[...] = h
    hb = h.astype(jnp.bfloat16)
    q_ref[...] = jnp.dot(hb, bdq_ref[...], preferred_element_type=jnp.float32) + b1_ref[...]
    n_ref[...] = jnp.dot(hb, bdn_ref[...], preferred_element_type=jnp.float32)


def _fuse_kernel(q_ref, x_ref, nb0_ref, nb1_ref, hb0_ref, hb1_ref, m2_ref,
                 wo_ref, w2_ref, r_ref, bo_ref, out_ref, z_scr):
    qv = q_ref[...]
    m2 = m2_ref[0]
    nwin = jnp.concatenate([nb0_ref[...], nb1_ref[...]], axis=0)

    m = jnp.full((T, H), -1e30, jnp.float32)
    for o in range(NO):
        t = jnp.maximum(qv + nwin[o:o + T], 0.0).astype(jnp.bfloat16)
        z = jnp.dot(t, w2_ref[...], preferred_element_type=jnp.float32)
        z = z + jnp.float32(-1e30) * m2[:, o:o + 1]
        z_scr[o] = z
        m = jnp.maximum(m, z)

    hwin = jnp.concatenate([hb0_ref[...], hb1_ref[...]], axis=0)
    acc = jnp.zeros((T, D), jnp.float32)
    l = jnp.zeros((T, H), jnp.float32)
    for o in range(NO):
        e = jnp.exp(z_scr[o] - m)
        pr = jnp.dot(e.astype(jnp.bfloat16), r_ref[...],
                     preferred_element_type=jnp.float32)
        acc = acc + pr * hwin[o:o + T]
        l = l + e

    lrep = jnp.dot(l, r_ref[...], preferred_element_type=jnp.float32)
    fused = acc / lrep
    out = jnp.dot(fused, wo_ref[...], preferred_element_type=jnp.float32)
    out_ref[...] = out + bo_ref[...] + x_ref[...]


def kernel(x, W_in, W1q, W1n, b1, W2, b2, W_out, b_out):
    inv_np, rep_np = _static_tables()
    x2 = x.reshape(S, D)
    eye = jnp.eye(H, dtype=jnp.float32)
    bdq = jnp.kron(eye, W1q).astype(jnp.bfloat16)
    bdn = jnp.kron(eye, W1n).astype(jnp.bfloat16)
    w2bd = jnp.kron(eye, W2).astype(jnp.bfloat16)
    b1t = jnp.tile(b1, H).reshape(1, D)
    bo = (b_out + b2[0] * 0.0).reshape(1, D)
    m2 = jnp.asarray(inv_np)
    rep = jnp.asarray(rep_np).astype(jnp.bfloat16)

    h, q, n = pl.pallas_call(
        _proj_kernel,
        grid=(NT,),
        in_specs=[
            pl.BlockSpec((T, D), lambda i: (i, 0)),
            pl.BlockSpec((D, D), lambda i: (0, 0)),
            pl.BlockSpec((D, D), lambda i: (0, 0)),
            pl.BlockSpec((D, D), lambda i: (0, 0)),
            pl.BlockSpec((1, D), lambda i: (0, 0)),
        ],
        out_specs=[
            pl.BlockSpec((T, D), lambda i: (i, 0)),
            pl.BlockSpec((T, D), lambda i: (i, 0)),
            pl.BlockSpec((T, D), lambda i: (i, 0)),
        ],
        out_shape=[jax.ShapeDtypeStruct((S, D), jnp.float32)] * 3,
    )(x2, W_in, bdq, bdn, b1t)

    h_pad = jnp.pad(h, ((W, S_PAD - S - W), (0, 0)))
    n_pad = jnp.pad(n, ((W, S_PAD - S - W), (0, 0)))

    out2 = pl.pallas_call(
        _fuse_kernel,
        grid=(NT,),
        in_specs=[
            pl.BlockSpec((T, D), lambda i: (i, 0)),
            pl.BlockSpec((T, D), lambda i: (i, 0)),
            pl.BlockSpec((T, D), lambda i: (i, 0)),
            pl.BlockSpec((T, D), lambda i: (i + 1, 0)),
            pl.BlockSpec((T, D), lambda i: (i, 0)),
            pl.BlockSpec((T, D), lambda i: (i + 1, 0)),
            pl.BlockSpec((1, T, NO), lambda i: (i, 0, 0)),
            pl.BlockSpec((D, D), lambda i: (0, 0)),
            pl.BlockSpec((D, H), lambda i: (0, 0)),
            pl.BlockSpec((H, D), lambda i: (0, 0)),
            pl.BlockSpec((1, D), lambda i: (0, 0)),
        ],
        out_specs=pl.BlockSpec((T, D), lambda i: (i, 0)),
        out_shape=jax.ShapeDtypeStruct((S, D), jnp.float32),
        scratch_shapes=[
            pltpu.VMEM((NO, T, H), jnp.float32),
        ],
    )(q, x2, n_pad, n_pad, h_pad, h_pad, m2, W_out, w2bd, rep, bo)

    return out2.reshape(B, S, D)
